# baseline (speedup 1.0000x reference)
import functools

import jax
import jax.numpy as jnp
from jax import lax
from jax.experimental import pallas as pl
from jax.experimental.pallas import tpu as pltpu
from jax.experimental.pallas import tpu_sc as plsc

N = 10000
H = 128
E = 320000
CHUNK = 128
IB = 32
NC, NS = 2, 16
_EGRAN = CHUNK * NS * IB
E_PAD = ((E + _EGRAN - 1) // _EGRAN) * _EGRAN
TOT_CHUNKS = E_PAD // CHUNK
CPT = TOT_CHUNKS // NS
RPT = 624
RB = 24
NBLK = RPT // RB
TAIL_BASE = NS * RPT
TAIL = N - TAIL_BASE

_MESH = plsc.VectorSubcoreMesh(
    core_axis_name="c", subcore_axis_name="s", num_cores=NC, num_subcores=NS
)

_SCRATCH = [
    pltpu.VMEM((IB, CHUNK), jnp.int32),
    pltpu.VMEM((IB, CHUNK), jnp.int32),
    pltpu.VMEM((IB, CHUNK), jnp.float32),
    pltpu.VMEM((CHUNK, H), jnp.float32),
    pltpu.VMEM((RB, H), jnp.float32),
    pltpu.VMEM((8, H), jnp.float32),
    pltpu.VMEM_SHARED((N, H), jnp.float32),
    pltpu.SemaphoreType.DMA,
]


def _chain(tab1, mid, out, bias_hbm, src_r, dst_r, w_r,
           src_v, dst_v, w_v, gbuf, rowb, bias_v, acc, sem, sid):
    base = sid * CPT

    def zero_rowb():
        zero = jnp.zeros((16,), jnp.float32)

        def zrow(i, _):
            for v in range(H // 16):
                rowb[i, pl.ds(v * 16, 16)] = zero
            return 0

        lax.fori_loop(0, RB, zrow, 0)

    def zero_acc():
        zero_rowb()

        def zblk(k, _):
            pltpu.sync_copy(rowb, acc.at[pl.ds(sid * RPT + k * RB, RB)])
            return 0

        lax.fori_loop(0, NBLK, zblk, 0)

        @pl.when(sid == 0)
        def _():
            pltpu.sync_copy(rowb.at[pl.ds(0, TAIL)],
                            acc.at[pl.ds(TAIL_BASE, TAIL)])

    def scatter_chunks(tab):
        def group_body(g, _):
            gb = base + g * IB
            pltpu.sync_copy(src_r.at[pl.ds(gb, IB)], src_v)
            pltpu.sync_copy(dst_r.at[pl.ds(gb, IB)], dst_v)
            pltpu.sync_copy(w_r.at[pl.ds(gb, IB)], w_v)

            def chunk_body(jj, _):
                pltpu.async_copy(tab.at[src_v.at[jj]], gbuf, sem).wait()

                def vgroup(gg, _):
                    wv16 = w_v[jj, pl.ds(gg * 16, 16)]
                    for l in range(16):
                        ws = jnp.full((16,), wv16[l], jnp.float32)
                        e = gg * 16 + l
                        for v in range(H // 16):
                            sl = pl.ds(v * 16, 16)
                            gbuf[e, sl] = gbuf[e, sl] * ws
                    return 0

                lax.fori_loop(0, CHUNK // 16, vgroup, 0)
                pltpu.sync_copy(gbuf, acc.at[dst_v.at[jj]], add=True)
                return 0

            lax.fori_loop(0, IB, chunk_body, 0)
            return 0

        lax.fori_loop(0, CPT // IB, group_body, 0)

    def flush(out_ref, relu_bias):
        def emit(row, nrows):
            pltpu.sync_copy(acc.at[pl.ds(row, nrows)],
                            rowb.at[pl.ds(0, nrows)])
            if relu_bias:
                def frow(i, _):
                    for v in range(H // 16):
                        sl = pl.ds(v * 16, 16)
                        rowb[i, sl] = jnp.maximum(
                            rowb[i, sl] + bias_v[0, sl], 0.0)
                    return 0

                lax.fori_loop(0, nrows, frow, 0)
            pltpu.sync_copy(rowb.at[pl.ds(0, nrows)],
                            out_ref.at[pl.ds(row, nrows)])

        def fblk(k, _):
            emit(sid * RPT + k * RB, RB)
            return 0

        lax.fori_loop(0, NBLK, fblk, 0)

        @pl.when(sid == 0)
        def _():
            emit(TAIL_BASE, TAIL)

    pltpu.sync_copy(bias_hbm, bias_v.at[pl.ds(0, 1)])
    zero_acc()
    plsc.subcore_barrier()
    scatter_chunks(tab1)
    plsc.subcore_barrier()
    flush(mid, True)
    plsc.subcore_barrier()
    zero_acc()
    plsc.subcore_barrier()
    scatter_chunks(mid)
    plsc.subcore_barrier()
    flush(out, False)


@functools.partial(
    pl.kernel,
    out_type=(
        jax.ShapeDtypeStruct((N, H), jnp.float32),
        jax.ShapeDtypeStruct((N, H), jnp.float32),
        jax.ShapeDtypeStruct((N, H), jnp.float32),
        jax.ShapeDtypeStruct((N, H), jnp.float32),
    ),
    mesh=_MESH,
    scratch_types=_SCRATCH,
)
def _sc_l1(p_r, p_u, src_r, dst_r, w_r, br1, bu1,
           h1_r, h1_u, rr_o, ru_o,
           src_v, dst_v, w_v, gbuf, rowb, bias_v, acc, sem):
    cid = lax.axis_index("c")
    sid = lax.axis_index("s")

    @pl.when(cid == 0)
    def _():
        _chain(p_r, h1_r, rr_o, br1, src_r, dst_r, w_r,
               src_v, dst_v, w_v, gbuf, rowb, bias_v, acc, sem, sid)

    @pl.when(cid == 1)
    def _():
        _chain(p_u, h1_u, ru_o, bu1, src_r, dst_r, w_r,
               src_v, dst_v, w_v, gbuf, rowb, bias_v, acc, sem, sid)


@functools.partial(
    pl.kernel,
    out_type=(
        jax.ShapeDtypeStruct((N, H), jnp.float32),
        jax.ShapeDtypeStruct((N, H), jnp.float32),
    ),
    mesh=_MESH,
    scratch_types=_SCRATCH,
)
def _sc_l2(t, src_r, dst_r, w_r, bc1,
           h1_c, rc_o,
           src_v, dst_v, w_v, gbuf, rowb, bias_v, acc, sem):
    cid = lax.axis_index("c")
    sid = lax.axis_index("s")

    @pl.when(cid == 0)
    def _():
        _chain(t, h1_c, rc_o, bc1, src_r, dst_r, w_r,
               src_v, dst_v, w_v, gbuf, rowb, bias_v, acc, sem, sid)


_RBK = 1000


def _row_spec(width=H):
    return pl.BlockSpec((_RBK, width), lambda i: (i, 0))


def _full_spec(r, c):
    return pl.BlockSpec((r, c), lambda i: (0, 0))


def _dot(a, b):
    return jnp.dot(a, b, preferred_element_type=jnp.float32)


def _tc0_body(x, h, wrx, wrh, wux, wuh, wcx, pr_o, pu_o, pcx_o):
    pr_o[:] = _dot(x[:], wrx[:]) + _dot(h[:], wrh[:])
    pu_o[:] = _dot(x[:], wux[:]) + _dot(h[:], wuh[:])
    pcx_o[:] = _dot(x[:], wcx[:])


def _tc0(x, h, wrx, wrh, wux, wuh, wcx):
    return pl.pallas_call(
        _tc0_body,
        grid=(N // _RBK,),
        in_specs=[_row_spec(), _row_spec()] + [_full_spec(H, H)] * 5,
        out_specs=[_row_spec()] * 3,
        out_shape=[jax.ShapeDtypeStruct((N, H), jnp.float32)] * 3,
    )(x, h, wrx, wrh, wux, wuh, wcx)


def _tc1_body(rr, ru, wr2, wu2, br2, bu2, h, pcx, wc1h, t_o, u_o):
    r = jax.nn.sigmoid(_dot(rr[:], wr2[:]) + br2[:])
    u = jax.nn.sigmoid(_dot(ru[:], wu2[:]) + bu2[:])
    t_o[:] = _dot(r * h[:], wc1h[:]) + pcx[:]
    u_o[:] = u


def _tc1(rr, ru, wr2, wu2, br2, bu2, h, pcx, wc1h):
    return pl.pallas_call(
        _tc1_body,
        grid=(N // _RBK,),
        in_specs=[_row_spec(), _row_spec(), _full_spec(H, H), _full_spec(H, H),
                  _full_spec(1, H), _full_spec(1, H), _row_spec(),
                  _row_spec(), _full_spec(H, H)],
        out_specs=[_row_spec(), _row_spec()],
        out_shape=[jax.ShapeDtypeStruct((N, H), jnp.float32)] * 2,
    )(rr, ru, wr2, wu2, br2, bu2, h, pcx, wc1h)


def _tc2_body(rc, wc2, bc2, u, h, o):
    c = jnp.tanh(_dot(rc[:], wc2[:]) + bc2[:])
    uu = u[:]
    o[:] = uu * h[:] + (1.0 - uu) * c


def _tc2(rc, wc2, bc2, u, h):
    return pl.pallas_call(
        _tc2_body,
        grid=(N // _RBK,),
        in_specs=[_row_spec(), _full_spec(H, H), _full_spec(1, H),
                  _row_spec(), _row_spec()],
        out_specs=_row_spec(),
        out_shape=jax.ShapeDtypeStruct((N, H), jnp.float32),
    )(rc, wc2, bc2, u, h)


def kernel(x, edge_index, edge_weight, h,
           Wr1, br1, Wr2, br2, Wu1, bu1, Wu2, bu2, Wc1, bc1, Wc2, bc2):
    spread = jnp.arange(E_PAD, dtype=jnp.int32) % N
    src = lax.dynamic_update_slice(spread, edge_index[0], (0,))
    dst = lax.dynamic_update_slice(spread, edge_index[1], (0,))
    w = lax.dynamic_update_slice(jnp.zeros((E_PAD,), jnp.float32),
                                 edge_weight, (0,))
    src = src.reshape(TOT_CHUNKS, CHUNK)
    dst = dst.reshape(TOT_CHUNKS, CHUNK)
    w = w.reshape(TOT_CHUNKS, CHUNK)

    p_r, p_u, pcx = _tc0(x, h, Wr1[:H], Wr1[H:], Wu1[:H], Wu1[H:], Wc1[:H])
    _h1r, _h1u, r_r, r_u = _sc_l1(p_r, p_u, src, dst, w,
                                  br1.reshape(1, H), bu1.reshape(1, H))
    t, u = _tc1(r_r, r_u, Wr2, Wu2, br2.reshape(1, H), bu2.reshape(1, H),
                h, pcx, Wc1[H:])
    _h1c, r_c = _sc_l2(t, src, dst, w, bc1.reshape(1, H))
    return _tc2(r_c, Wc2, bc2.reshape(1, H), u, h)


# speedup vs baseline: 1.5816x; 1.5816x over previous
import functools

import jax
import jax.numpy as jnp
from jax import lax
from jax.experimental import pallas as pl
from jax.experimental.pallas import tpu as pltpu
from jax.experimental.pallas import tpu_sc as plsc

N = 10000
H = 128
E = 320000
CHUNK = 128
IB = 32
NC, NS = 2, 16
_EGRAN = CHUNK * NS * IB
E_PAD = ((E + _EGRAN - 1) // _EGRAN) * _EGRAN
TOT_CHUNKS = E_PAD // CHUNK
CPT = TOT_CHUNKS // NS
RPT = 624
RB = 24
NBLK = RPT // RB
TAIL_BASE = NS * RPT
TAIL = N - TAIL_BASE

_MESH = plsc.VectorSubcoreMesh(
    core_axis_name="c", subcore_axis_name="s", num_cores=NC, num_subcores=NS
)

_SCRATCH = [
    pltpu.VMEM((IB, CHUNK), jnp.int32),
    pltpu.VMEM((IB, CHUNK), jnp.int32),
    pltpu.VMEM((IB, CHUNK), jnp.float32),
    pltpu.VMEM((CHUNK, H), jnp.float32),
    pltpu.VMEM((CHUNK, H), jnp.float32),
    pltpu.VMEM((RB, H), jnp.float32),
    pltpu.VMEM((8, H), jnp.float32),
    pltpu.VMEM_SHARED((N, H), jnp.float32),
    pltpu.SemaphoreType.DMA,
    pltpu.SemaphoreType.DMA,
]


def _chain(tab1, mid, out, bias_hbm, src_r, dst_r, w_r,
           src_v, dst_v, w_v, gbufa, gbufb, rowb, bias_v, acc,
           sem_g, sem_s, sid):
    base = sid * CPT

    def zero_rowb():
        zero = jnp.zeros((16,), jnp.float32)

        def zrow(i, _):
            for v in range(H // 16):
                rowb[i, pl.ds(v * 16, 16)] = zero
            return 0

        lax.fori_loop(0, RB, zrow, 0)

    def zero_acc():
        zero_rowb()

        def zblk(k, _):
            pltpu.sync_copy(rowb, acc.at[pl.ds(sid * RPT + k * RB, RB)])
            return 0

        lax.fori_loop(0, NBLK, zblk, 0)

        @pl.when(sid == 0)
        def _():
            pltpu.sync_copy(rowb.at[pl.ds(0, TAIL)],
                            acc.at[pl.ds(TAIL_BASE, TAIL)])

    def scale(buf, jj):
        def vgroup(gg, _):
            wv16 = w_v[jj, pl.ds(gg * 16, 16)]
            for l in range(16):
                ws = jnp.full((16,), wv16[l], jnp.float32)
                e = gg * 16 + l
                for v in range(H // 16):
                    sl = pl.ds(v * 16, 16)
                    buf[e, sl] = buf[e, sl] * ws
            return 0

        lax.fori_loop(0, CHUNK // 16, vgroup, 0)

    def scatter_chunks(tab):
        def group_body(g, _):
            gb = base + g * IB
            pltpu.sync_copy(src_r.at[pl.ds(gb, IB)], src_v)
            pltpu.sync_copy(dst_r.at[pl.ds(gb, IB)], dst_v)
            pltpu.sync_copy(w_r.at[pl.ds(gb, IB)], w_v)
            pltpu.async_copy(tab.at[src_v.at[0]], gbufa, sem_g)

            def pair_body(p, _):
                j0 = 2 * p
                j1 = 2 * p + 1
                pltpu.make_async_copy(tab.at[src_v.at[j0]],
                                      gbufa, sem_g).wait()

                @pl.when(p > 0)
                def _():
                    pltpu.make_async_copy(
                        gbufb, acc.at[dst_v.at[j0 - 1]], sem_s).wait()

                pltpu.async_copy(tab.at[src_v.at[j1]], gbufb, sem_g)
                scale(gbufa, j0)
                pltpu.async_copy(gbufa, acc.at[dst_v.at[j0]], sem_s,
                                 add=True)
                pltpu.make_async_copy(tab.at[src_v.at[j1]],
                                      gbufb, sem_g).wait()
                pltpu.make_async_copy(gbufa, acc.at[dst_v.at[j0]],
                                      sem_s).wait()

                @pl.when(p + 1 < IB // 2)
                def _():
                    pltpu.async_copy(tab.at[src_v.at[j1 + 1]],
                                     gbufa, sem_g)

                scale(gbufb, j1)
                pltpu.async_copy(gbufb, acc.at[dst_v.at[j1]], sem_s,
                                 add=True)
                return 0

            lax.fori_loop(0, IB // 2, pair_body, 0)
            pltpu.make_async_copy(gbufb, acc.at[dst_v.at[IB - 1]],
                                  sem_s).wait()
            return 0

        lax.fori_loop(0, CPT // IB, group_body, 0)

    def flush(out_ref, relu_bias):
        def emit(row, nrows):
            pltpu.sync_copy(acc.at[pl.ds(row, nrows)],
                            rowb.at[pl.ds(0, nrows)])
            if relu_bias:
                def frow(i, _):
                    for v in range(H // 16):
                        sl = pl.ds(v * 16, 16)
                        rowb[i, sl] = jnp.maximum(
                            rowb[i, sl] + bias_v[0, sl], 0.0)
                    return 0

                lax.fori_loop(0, nrows, frow, 0)
            pltpu.sync_copy(rowb.at[pl.ds(0, nrows)],
                            out_ref.at[pl.ds(row, nrows)])

        def fblk(k, _):
            emit(sid * RPT + k * RB, RB)
            return 0

        lax.fori_loop(0, NBLK, fblk, 0)

        @pl.when(sid == 0)
        def _():
            emit(TAIL_BASE, TAIL)

    pltpu.sync_copy(bias_hbm, bias_v.at[pl.ds(0, 1)])
    zero_acc()
    plsc.subcore_barrier()
    scatter_chunks(tab1)
    plsc.subcore_barrier()
    flush(mid, True)
    plsc.subcore_barrier()
    zero_acc()
    plsc.subcore_barrier()
    scatter_chunks(mid)
    plsc.subcore_barrier()
    flush(out, False)


@functools.partial(
    pl.kernel,
    out_type=(
        jax.ShapeDtypeStruct((N, H), jnp.float32),
        jax.ShapeDtypeStruct((N, H), jnp.float32),
        jax.ShapeDtypeStruct((N, H), jnp.float32),
        jax.ShapeDtypeStruct((N, H), jnp.float32),
    ),
    mesh=_MESH,
    scratch_types=_SCRATCH,
)
def _sc_l1(p_r, p_u, src_r, dst_r, w_r, br1, bu1,
           h1_r, h1_u, rr_o, ru_o,
           src_v, dst_v, w_v, gbufa, gbufb, rowb, bias_v, acc,
           sem_g, sem_s):
    cid = lax.axis_index("c")
    sid = lax.axis_index("s")

    @pl.when(cid == 0)
    def _():
        _chain(p_r, h1_r, rr_o, br1, src_r, dst_r, w_r,
               src_v, dst_v, w_v, gbufa, gbufb, rowb, bias_v, acc,
               sem_g, sem_s, sid)

    @pl.when(cid == 1)
    def _():
        _chain(p_u, h1_u, ru_o, bu1, src_r, dst_r, w_r,
               src_v, dst_v, w_v, gbufa, gbufb, rowb, bias_v, acc,
               sem_g, sem_s, sid)


@functools.partial(
    pl.kernel,
    out_type=(
        jax.ShapeDtypeStruct((N, H), jnp.float32),
        jax.ShapeDtypeStruct((N, H), jnp.float32),
    ),
    mesh=_MESH,
    scratch_types=_SCRATCH,
)
def _sc_l2(t, src_r, dst_r, w_r, bc1,
           h1_c, rc_o,
           src_v, dst_v, w_v, gbufa, gbufb, rowb, bias_v, acc,
           sem_g, sem_s):
    cid = lax.axis_index("c")
    sid = lax.axis_index("s")

    @pl.when(cid == 0)
    def _():
        _chain(t, h1_c, rc_o, bc1, src_r, dst_r, w_r,
               src_v, dst_v, w_v, gbufa, gbufb, rowb, bias_v, acc,
               sem_g, sem_s, sid)


_RBK = 1000


def _row_spec(width=H):
    return pl.BlockSpec((_RBK, width), lambda i: (i, 0))


def _full_spec(r, c):
    return pl.BlockSpec((r, c), lambda i: (0, 0))


def _dot(a, b):
    return jnp.dot(a, b, preferred_element_type=jnp.float32)


def _tc0_body(x, h, wrx, wrh, wux, wuh, wcx, pr_o, pu_o, pcx_o):
    pr_o[:] = _dot(x[:], wrx[:]) + _dot(h[:], wrh[:])
    pu_o[:] = _dot(x[:], wux[:]) + _dot(h[:], wuh[:])
    pcx_o[:] = _dot(x[:], wcx[:])


def _tc0(x, h, wrx, wrh, wux, wuh, wcx):
    return pl.pallas_call(
        _tc0_body,
        grid=(N // _RBK,),
        in_specs=[_row_spec(), _row_spec()] + [_full_spec(H, H)] * 5,
        out_specs=[_row_spec()] * 3,
        out_shape=[jax.ShapeDtypeStruct((N, H), jnp.float32)] * 3,
    )(x, h, wrx, wrh, wux, wuh, wcx)


def _tc1_body(rr, ru, wr2, wu2, br2, bu2, h, pcx, wc1h, t_o, u_o):
    r = jax.nn.sigmoid(_dot(rr[:], wr2[:]) + br2[:])
    u = jax.nn.sigmoid(_dot(ru[:], wu2[:]) + bu2[:])
    t_o[:] = _dot(r * h[:], wc1h[:]) + pcx[:]
    u_o[:] = u


def _tc1(rr, ru, wr2, wu2, br2, bu2, h, pcx, wc1h):
    return pl.pallas_call(
        _tc1_body,
        grid=(N // _RBK,),
        in_specs=[_row_spec(), _row_spec(), _full_spec(H, H), _full_spec(H, H),
                  _full_spec(1, H), _full_spec(1, H), _row_spec(),
                  _row_spec(), _full_spec(H, H)],
        out_specs=[_row_spec(), _row_spec()],
        out_shape=[jax.ShapeDtypeStruct((N, H), jnp.float32)] * 2,
    )(rr, ru, wr2, wu2, br2, bu2, h, pcx, wc1h)


def _tc2_body(rc, wc2, bc2, u, h, o):
    c = jnp.tanh(_dot(rc[:], wc2[:]) + bc2[:])
    uu = u[:]
    o[:] = uu * h[:] + (1.0 - uu) * c


def _tc2(rc, wc2, bc2, u, h):
    return pl.pallas_call(
        _tc2_body,
        grid=(N // _RBK,),
        in_specs=[_row_spec(), _full_spec(H, H), _full_spec(1, H),
                  _row_spec(), _row_spec()],
        out_specs=_row_spec(),
        out_shape=jax.ShapeDtypeStruct((N, H), jnp.float32),
    )(rc, wc2, bc2, u, h)


def kernel(x, edge_index, edge_weight, h,
           Wr1, br1, Wr2, br2, Wu1, bu1, Wu2, bu2, Wc1, bc1, Wc2, bc2):
    spread = jnp.arange(E_PAD, dtype=jnp.int32) % N
    src = lax.dynamic_update_slice(spread, edge_index[0], (0,))
    dst = lax.dynamic_update_slice(spread, edge_index[1], (0,))
    w = lax.dynamic_update_slice(jnp.zeros((E_PAD,), jnp.float32),
                                 edge_weight, (0,))
    src = src.reshape(TOT_CHUNKS, CHUNK)
    dst = dst.reshape(TOT_CHUNKS, CHUNK)
    w = w.reshape(TOT_CHUNKS, CHUNK)

    p_r, p_u, pcx = _tc0(x, h, Wr1[:H], Wr1[H:], Wu1[:H], Wu1[H:], Wc1[:H])
    _h1r, _h1u, r_r, r_u = _sc_l1(p_r, p_u, src, dst, w,
                                  br1.reshape(1, H), bu1.reshape(1, H))
    t, u = _tc1(r_r, r_u, Wr2, Wu2, br2.reshape(1, H), bu2.reshape(1, H),
                h, pcx, Wc1[H:])
    _h1c, r_c = _sc_l2(t, src, dst, w, bc1.reshape(1, H))
    return _tc2(r_c, Wc2, bc2.reshape(1, H), u, h)


# speedup vs baseline: 1.9877x; 1.2567x over previous
import functools

import jax
import jax.numpy as jnp
from jax import lax
from jax.experimental import pallas as pl
from jax.experimental.pallas import tpu as pltpu
from jax.experimental.pallas import tpu_sc as plsc

N = 10000
H = 128
E = 320000
CHUNK = 128
IB = 32
IBH = 16
NC, NS = 2, 16
_EGRAN = CHUNK * NS * IB
E_PAD = ((E + _EGRAN - 1) // _EGRAN) * _EGRAN
TOT_CHUNKS = E_PAD // CHUNK
CPT = TOT_CHUNKS // NS
CPTH = TOT_CHUNKS // (NS * NC)
RPT = 624
RB = 24
NBLK = RPT // RB
TAIL_BASE = NS * RPT
TAIL = N - TAIL_BASE

_MESH = plsc.VectorSubcoreMesh(
    core_axis_name="c", subcore_axis_name="s", num_cores=NC, num_subcores=NS
)


def _zero_rowb(rowb):
    zero = jnp.zeros((16,), jnp.float32)

    def zrow(i, _):
        for v in range(H // 16):
            rowb[i, pl.ds(v * 16, 16)] = zero
        return 0

    lax.fori_loop(0, RB, zrow, 0)


def _zero_acc(acc, rowb, sid):
    _zero_rowb(rowb)

    def zblk(k, _):
        pltpu.sync_copy(rowb, acc.at[pl.ds(sid * RPT + k * RB, RB)])
        return 0

    lax.fori_loop(0, NBLK, zblk, 0)

    @pl.when(sid == 0)
    def _():
        pltpu.sync_copy(rowb.at[pl.ds(0, TAIL)],
                        acc.at[pl.ds(TAIL_BASE, TAIL)])


def _scale(w_v, buf, jj):
    def vgroup(gg, _):
        wv16 = w_v[jj, pl.ds(gg * 16, 16)]
        for l in range(16):
            ws = jnp.full((16,), wv16[l], jnp.float32)
            e = gg * 16 + l
            for v in range(H // 16):
                sl = pl.ds(v * 16, 16)
                buf[e, sl] = buf[e, sl] * ws
        return 0

    lax.fori_loop(0, CHUNK // 16, vgroup, 0)


def _scatter_chunks(tab, acc, base, cpt, ib,
                    src_r, dst_r, w_r, src_v, dst_v, w_v,
                    gbufa, gbufb, sem_g, sem_s):
    def group_body(g, _):
        gb = base + g * ib
        pltpu.sync_copy(src_r.at[pl.ds(gb, ib)], src_v)
        pltpu.sync_copy(dst_r.at[pl.ds(gb, ib)], dst_v)
        pltpu.sync_copy(w_r.at[pl.ds(gb, ib)], w_v)
        pltpu.async_copy(tab.at[src_v.at[0]], gbufa, sem_g)

        def pair_body(p, _):
            j0 = 2 * p
            j1 = 2 * p + 1
            pltpu.make_async_copy(tab.at[src_v.at[j0]], gbufa, sem_g).wait()

            @pl.when(p > 0)
            def _():
                pltpu.make_async_copy(
                    gbufb, acc.at[dst_v.at[j0 - 1]], sem_s).wait()

            pltpu.async_copy(tab.at[src_v.at[j1]], gbufb, sem_g)
            _scale(w_v, gbufa, j0)
            pltpu.async_copy(gbufa, acc.at[dst_v.at[j0]], sem_s, add=True)
            pltpu.make_async_copy(tab.at[src_v.at[j1]], gbufb, sem_g).wait()
            pltpu.make_async_copy(gbufa, acc.at[dst_v.at[j0]], sem_s).wait()

            @pl.when(p + 1 < ib // 2)
            def _():
                pltpu.async_copy(tab.at[src_v.at[j1 + 1]], gbufa, sem_g)

            _scale(w_v, gbufb, j1)
            pltpu.async_copy(gbufb, acc.at[dst_v.at[j1]], sem_s, add=True)
            return 0

        lax.fori_loop(0, ib // 2, pair_body, 0)
        pltpu.make_async_copy(gbufb, acc.at[dst_v.at[ib - 1]], sem_s).wait()
        return 0

    lax.fori_loop(0, cpt // ib, group_body, 0)


def _flush(acc, rowb, out_ref, sid, bias_v):
    def emit(row, nrows):
        pltpu.sync_copy(acc.at[pl.ds(row, nrows)], rowb.at[pl.ds(0, nrows)])
        if bias_v is not None:
            def frow(i, _):
                for v in range(H // 16):
                    sl = pl.ds(v * 16, 16)
                    rowb[i, sl] = jnp.maximum(
                        rowb[i, sl] + bias_v[0, sl], 0.0)
                return 0

            lax.fori_loop(0, nrows, frow, 0)
        pltpu.sync_copy(rowb.at[pl.ds(0, nrows)],
                        out_ref.at[pl.ds(row, nrows)])

    def fblk(k, _):
        emit(sid * RPT + k * RB, RB)
        return 0

    lax.fori_loop(0, NBLK, fblk, 0)

    @pl.when(sid == 0)
    def _():
        emit(TAIL_BASE, TAIL)


_SCRATCH = [
    pltpu.VMEM((IB, CHUNK), jnp.int32),
    pltpu.VMEM((IB, CHUNK), jnp.int32),
    pltpu.VMEM((IB, CHUNK), jnp.float32),
    pltpu.VMEM((CHUNK, H), jnp.float32),
    pltpu.VMEM((CHUNK, H), jnp.float32),
    pltpu.VMEM((RB, H), jnp.float32),
    pltpu.VMEM((8, H), jnp.float32),
    pltpu.VMEM_SHARED((N, H), jnp.float32),
    pltpu.SemaphoreType.DMA,
    pltpu.SemaphoreType.DMA,
]


def _chain(tab1, mid, out, bias_hbm, src_r, dst_r, w_r,
           src_v, dst_v, w_v, gbufa, gbufb, rowb, bias_v, acc,
           sem_g, sem_s, sid):
    base = sid * CPT

    def sc(tab):
        _scatter_chunks(tab, acc, base, CPT, IB, src_r, dst_r, w_r,
                        src_v, dst_v, w_v, gbufa, gbufb, sem_g, sem_s)

    pltpu.sync_copy(bias_hbm, bias_v.at[pl.ds(0, 1)])
    _zero_acc(acc, rowb, sid)
    plsc.subcore_barrier()
    sc(tab1)
    plsc.subcore_barrier()
    _flush(acc, rowb, mid, sid, bias_v)
    plsc.subcore_barrier()
    _zero_acc(acc, rowb, sid)
    plsc.subcore_barrier()
    sc(mid)
    plsc.subcore_barrier()
    _flush(acc, rowb, out, sid, None)


@functools.partial(
    pl.kernel,
    out_type=(
        jax.ShapeDtypeStruct((N, H), jnp.float32),
        jax.ShapeDtypeStruct((N, H), jnp.float32),
        jax.ShapeDtypeStruct((N, H), jnp.float32),
        jax.ShapeDtypeStruct((N, H), jnp.float32),
    ),
    mesh=_MESH,
    scratch_types=_SCRATCH,
)
def _sc_l1(p_r, p_u, src_r, dst_r, w_r, br1, bu1,
           h1_r, h1_u, rr_o, ru_o,
           src_v, dst_v, w_v, gbufa, gbufb, rowb, bias_v, acc,
           sem_g, sem_s):
    cid = lax.axis_index("c")
    sid = lax.axis_index("s")

    @pl.when(cid == 0)
    def _():
        _chain(p_r, h1_r, rr_o, br1, src_r, dst_r, w_r,
               src_v, dst_v, w_v, gbufa, gbufb, rowb, bias_v, acc,
               sem_g, sem_s, sid)

    @pl.when(cid == 1)
    def _():
        _chain(p_u, h1_u, ru_o, bu1, src_r, dst_r, w_r,
               src_v, dst_v, w_v, gbufa, gbufb, rowb, bias_v, acc,
               sem_g, sem_s, sid)


_SCRATCH_H = [
    pltpu.VMEM((IBH, CHUNK), jnp.int32),
    pltpu.VMEM((IBH, CHUNK), jnp.int32),
    pltpu.VMEM((IBH, CHUNK), jnp.float32),
    pltpu.VMEM((CHUNK, H), jnp.float32),
    pltpu.VMEM((CHUNK, H), jnp.float32),
    pltpu.VMEM((RB, H), jnp.float32),
    pltpu.VMEM_SHARED((N, H), jnp.float32),
    pltpu.SemaphoreType.DMA,
    pltpu.SemaphoreType.DMA,
]


@functools.partial(
    pl.kernel,
    out_type=(
        jax.ShapeDtypeStruct((N, H), jnp.float32),
        jax.ShapeDtypeStruct((N, H), jnp.float32),
    ),
    mesh=_MESH,
    scratch_types=_SCRATCH_H,
)
def _sc_half(tab, src_r, dst_r, w_r, out0, out1,
             src_v, dst_v, w_v, gbufa, gbufb, rowb, acc, sem_g, sem_s):
    cid = lax.axis_index("c")
    sid = lax.axis_index("s")
    base = (sid * NC + cid) * CPTH

    _zero_acc(acc, rowb, sid)
    plsc.subcore_barrier()
    _scatter_chunks(tab, acc, base, CPTH, IBH, src_r, dst_r, w_r,
                    src_v, dst_v, w_v, gbufa, gbufb, sem_g, sem_s)
    plsc.subcore_barrier()

    @pl.when(cid == 0)
    def _():
        _flush(acc, rowb, out0, sid, None)

    @pl.when(cid == 1)
    def _():
        _flush(acc, rowb, out1, sid, None)


_RBK = 1000


def _row_spec(width=H):
    return pl.BlockSpec((_RBK, width), lambda i: (i, 0))


def _full_spec(r, c):
    return pl.BlockSpec((r, c), lambda i: (0, 0))


def _dot(a, b):
    return jnp.dot(a, b, preferred_element_type=jnp.float32)


def _tc0_body(x, h, wrx, wrh, wux, wuh, wcx, pr_o, pu_o, pcx_o):
    pr_o[:] = _dot(x[:], wrx[:]) + _dot(h[:], wrh[:])
    pu_o[:] = _dot(x[:], wux[:]) + _dot(h[:], wuh[:])
    pcx_o[:] = _dot(x[:], wcx[:])


def _tc0(x, h, wrx, wrh, wux, wuh, wcx):
    return pl.pallas_call(
        _tc0_body,
        grid=(N // _RBK,),
        in_specs=[_row_spec(), _row_spec()] + [_full_spec(H, H)] * 5,
        out_specs=[_row_spec()] * 3,
        out_shape=[jax.ShapeDtypeStruct((N, H), jnp.float32)] * 3,
    )(x, h, wrx, wrh, wux, wuh, wcx)


def _tc1_body(rr, ru, wr2, wu2, br2, bu2, h, pcx, wc1h, t_o, u_o):
    r = jax.nn.sigmoid(_dot(rr[:], wr2[:]) + br2[:])
    u = jax.nn.sigmoid(_dot(ru[:], wu2[:]) + bu2[:])
    t_o[:] = _dot(r * h[:], wc1h[:]) + pcx[:]
    u_o[:] = u


def _tc1(rr, ru, wr2, wu2, br2, bu2, h, pcx, wc1h):
    return pl.pallas_call(
        _tc1_body,
        grid=(N // _RBK,),
        in_specs=[_row_spec(), _row_spec(), _full_spec(H, H), _full_spec(H, H),
                  _full_spec(1, H), _full_spec(1, H), _row_spec(),
                  _row_spec(), _full_spec(H, H)],
        out_specs=[_row_spec(), _row_spec()],
        out_shape=[jax.ShapeDtypeStruct((N, H), jnp.float32)] * 2,
    )(rr, ru, wr2, wu2, br2, bu2, h, pcx, wc1h)


def _tcr_body(q0, q1, b, o):
    o[:] = jnp.maximum(q0[:] + q1[:] + b[:], 0.0)


def _tcr(q0, q1, b):
    return pl.pallas_call(
        _tcr_body,
        grid=(N // _RBK,),
        in_specs=[_row_spec(), _row_spec(), _full_spec(1, H)],
        out_specs=_row_spec(),
        out_shape=jax.ShapeDtypeStruct((N, H), jnp.float32),
    )(q0, q1, b)


def _tc2_body(rc0, rc1, wc2, bc2, u, h, o):
    c = jnp.tanh(_dot(rc0[:] + rc1[:], wc2[:]) + bc2[:])
    uu = u[:]
    o[:] = uu * h[:] + (1.0 - uu) * c


def _tc2(rc0, rc1, wc2, bc2, u, h):
    return pl.pallas_call(
        _tc2_body,
        grid=(N // _RBK,),
        in_specs=[_row_spec(), _row_spec(), _full_spec(H, H),
                  _full_spec(1, H), _row_spec(), _row_spec()],
        out_specs=_row_spec(),
        out_shape=jax.ShapeDtypeStruct((N, H), jnp.float32),
    )(rc0, rc1, wc2, bc2, u, h)


def kernel(x, edge_index, edge_weight, h,
           Wr1, br1, Wr2, br2, Wu1, bu1, Wu2, bu2, Wc1, bc1, Wc2, bc2):
    spread = jnp.arange(E_PAD, dtype=jnp.int32) % N
    src = lax.dynamic_update_slice(spread, edge_index[0], (0,))
    dst = lax.dynamic_update_slice(spread, edge_index[1], (0,))
    w = lax.dynamic_update_slice(jnp.zeros((E_PAD,), jnp.float32),
                                 edge_weight, (0,))
    src = src.reshape(TOT_CHUNKS, CHUNK)
    dst = dst.reshape(TOT_CHUNKS, CHUNK)
    w = w.reshape(TOT_CHUNKS, CHUNK)

    p_r, p_u, pcx = _tc0(x, h, Wr1[:H], Wr1[H:], Wu1[:H], Wu1[H:], Wc1[:H])
    _h1r, _h1u, r_r, r_u = _sc_l1(p_r, p_u, src, dst, w,
                                  br1.reshape(1, H), bu1.reshape(1, H))
    t, u = _tc1(r_r, r_u, Wr2, Wu2, br2.reshape(1, H), bu2.reshape(1, H),
                h, pcx, Wc1[H:])
    q0, q1 = _sc_half(t, src, dst, w)
    h1_c = _tcr(q0, q1, bc1.reshape(1, H))
    rc0, rc1 = _sc_half(h1_c, src, dst, w)
    return _tc2(rc0, rc1, Wc2, bc2.reshape(1, H), u, h)
